# Initial kernel scaffold; baseline (speedup 1.0000x reference)
import jax
import jax.numpy as jnp
from jax.experimental import pallas as pl


def kernel(logits, labels):
    raise NotImplementedError("write your pallas kernel here")


# speedup vs baseline: 1.8058x; 1.8058x over previous
import functools

import jax
import jax.numpy as jnp
from jax.experimental import pallas as pl
from jax.experimental.pallas import tpu as pltpu

ALPHA = 5.0
EPS = 1e-8


def _cosine_loss_kernel(labels_ref, x_ref, out_ref, acc_sumsq, acc_g,
                        *, n_rows, n_cols, block_cols, n_blocks):
    cb = pl.program_id(0)

    @pl.when(cb == 0)
    def _init():
        acc_sumsq[...] = jnp.zeros_like(acc_sumsq)
        acc_g[...] = jnp.zeros_like(acc_g)

    x = x_ref[...]
    col = cb * block_cols + jax.lax.broadcasted_iota(
        jnp.int32, (n_rows, block_cols), 1)
    valid = col < n_cols
    xm = jnp.where(valid, x, 0.0)
    acc_sumsq[...] += jnp.sum(xm * xm, axis=1, keepdims=True)
    lbl = labels_ref[...]
    acc_g[...] += jnp.sum(jnp.where(col == lbl, xm, 0.0), axis=1,
                          keepdims=True)

    @pl.when(cb == n_blocks - 1)
    def _finish():
        norm = jnp.sqrt(acc_sumsq[...])
        s = acc_g[...] / jnp.maximum(norm, EPS)
        loss_terms = (1.0 - s) / (1.0 + s) * ALPHA
        out_ref[0, 0] = jnp.sum(loss_terms) / n_rows


def kernel(logits, labels):
    n_rows, n_cols = logits.shape
    block_cols = 2048
    n_blocks = pl.cdiv(n_cols, block_cols)
    labels2 = labels.astype(jnp.int32).reshape(n_rows, 1)

    out = pl.pallas_call(
        functools.partial(
            _cosine_loss_kernel, n_rows=n_rows, n_cols=n_cols,
            block_cols=block_cols, n_blocks=n_blocks),
        grid=(n_blocks,),
        in_specs=[
            pl.BlockSpec((n_rows, 1), lambda cb: (0, 0)),
            pl.BlockSpec((n_rows, block_cols), lambda cb: (0, cb)),
        ],
        out_specs=pl.BlockSpec(
            (1, 1), lambda cb: (0, 0), memory_space=pltpu.SMEM),
        out_shape=jax.ShapeDtypeStruct((1, 1), jnp.float32),
        scratch_shapes=[
            pltpu.VMEM((n_rows, 1), jnp.float32),
            pltpu.VMEM((n_rows, 1), jnp.float32),
        ],
    )(labels2, logits)
    return out[0, 0]
